# baseline (speedup 1.0000x reference)
import jax
import jax.numpy as jnp
from jax.experimental import pallas as pl
from jax.experimental.pallas import tpu as pltpu

B = 8
C = 85
NT = 64
SIZES = (8.0, 16.0, 32.0, 64.0, 128.0)
GRIDS = (128, 64, 32, 16, 8)
IMG = 1024.0
CCHUNK = 8

def _mk_level(li, g, lrow0, rpb):
    s = SIZES[li]
    return dict(
        li=li,
        g=g,
        lrow0=lrow0,
        rpb=rpb,
        s=s,
        stride=IMG / g,
        lo=0.0 if li == 0 else s * 4.0,
        hi=float("inf") if li == 4 else s * 8.0,
        ncells=g * g,
    )


_NSTEPS = 11


def _level_body(t_ref, p_ref, fidx_ref, fcnt_ref, lstep, seg):
    f32 = jnp.float32
    li = seg["li"]
    g = seg["g"]
    rpb = seg["rpb"]
    r0 = seg["lrow0"]
    s = seg["s"]
    stride = seg["stride"]
    lo = seg["lo"]
    hi = seg["hi"]
    log2g = g.bit_length() - 1

    @pl.when(lstep == 0)
    def _():
        cnt = jnp.int32(0)
        for t in range(NT):
            w = t_ref[0, t, 4] - t_ref[0, t, 2]
            h = t_ref[0, t, 5] - t_ref[0, t, 3]
            mwh = jnp.maximum(w, h)
            feas = None
            if lo > 0.0:
                feas = mwh > (lo - 1.0)
            if hi != float("inf"):
                c2 = mwh * 0.5 < (hi + 1.0)
                feas = c2 if feas is None else (feas & c2)
            fidx_ref[li, cnt] = jnp.int32(t)
            cnt = cnt + (jnp.int32(1) if feas is None else feas.astype(jnp.int32))
        fcnt_ref[li] = cnt

    rowi = jax.lax.broadcasted_iota(jnp.int32, (rpb, 128), 0)
    lane = jax.lax.broadcasted_iota(jnp.int32, (rpb, 128), 1)
    cflat = (lstep * rpb + rowi) * 128 + lane
    ix = (cflat & (g - 1)).astype(f32)
    iy = jax.lax.shift_right_logical(cflat, log2g).astype(f32)
    x = (ix + 0.5) * stride
    y = (iy + 0.5) * stride

    radius = s * 2.0
    shape = (rpb, 128)

    enc0 = t_ref[0, 0, 0] * 256.0 + t_ref[0, 0, 1]
    init = (
        jnp.zeros(shape, f32),
        jnp.full(shape, enc0, f32),
        jnp.full(shape, t_ref[0, 0, 2], f32),
        jnp.full(shape, t_ref[0, 0, 3], f32),
        jnp.full(shape, t_ref[0, 0, 4], f32),
        jnp.full(shape, t_ref[0, 0, 5], f32),
    )

    def tbody(k, carry):
        best, enc, ax0, ay0, ax1, ay1 = carry
        t = fidx_ref[li, k]
        nb = t_ref[0, t, 0]
        cl = t_ref[0, t, 1]
        x0 = t_ref[0, t, 2]
        y0 = t_ref[0, t, 3]
        x1 = t_ref[0, t, 4]
        y1 = t_ref[0, t, 5]
        l = x - x0
        tt = y - y0
        r = x1 - x
        bb = y1 - y
        omin = jnp.minimum(jnp.minimum(l, tt), jnp.minimum(r, bb))
        omax = jnp.maximum(jnp.maximum(l, tt), jnp.maximum(r, bb))
        cxb = (x0 + x1) / 2.0
        cyb = (y0 + y1) / 2.0
        cmax = jnp.maximum(jnp.abs(x - cxb), jnp.abs(y - cyb))
        jc = (omin > 0.0) & (cmax < radius)
        if lo > 0.0:
            jc = jc & (omax > lo)
        if hi != float("inf"):
            jc = jc & (omax < hi)
        area = (l + r) * (tt + bb)
        score = jnp.where(jc, 1e8 - area, 0.0)
        upd = score > best
        best = jnp.where(upd, score, best)
        enc = jnp.where(upd, nb * 256.0 + cl, enc)
        ax0 = jnp.where(upd, x0, ax0)
        ay0 = jnp.where(upd, y0, ay0)
        ax1 = jnp.where(upd, x1, ax1)
        ay1 = jnp.where(upd, y1, ay1)
        return (best, enc, ax0, ay0, ax1, ay1)

    best, enc, sx0, sy0, sx1, sy1 = jax.lax.fori_loop(
        0, fcnt_ref[li], tbody, init
    )
    keep = best > 0.0
    benc = enc.astype(jnp.int32)
    bsel = jax.lax.shift_right_logical(benc, 8)
    csel = benc & 255
    lsel = x - sx0
    tsel = y - sy0
    rsel = sx1 - x
    bsel_f = sy1 - y
    gx = lsel + sx0
    gy = tsel + sy0
    lr_min = jnp.minimum(lsel, rsel)
    lr_max = jnp.maximum(lsel, rsel)
    tb_min = jnp.minimum(tsel, bsel_f)
    tb_max = jnp.maximum(tsel, bsel_f)
    tcnt = jnp.sqrt(lr_min * tb_min / (lr_max * tb_max + 1e-10))
    tx0 = gx - lsel
    ty0 = gy - tsel
    tx1 = gx + rsel
    ty1 = gy + bsel_f

    pid = jnp.where(keep, bsel * C + (csel + 5), -1)

    facc = jnp.zeros((rpb, 128), f32)
    xpos = jnp.zeros((rpb, 128), f32)
    ps = [jnp.zeros((rpb, 128), f32) for _ in range(5)]
    ciota = jax.lax.broadcasted_iota(jnp.int32, (CCHUNK, 1, 1), 0)
    for b in range(B):
        bm = bsel == b
        for c in range(5):
            ps[c] = ps[c] + jnp.where(bm, p_ref[b, c, r0 : r0 + rpb, :], 0.0)
        pid_b = (pid - b * C)[None, :, :]
        for c0 in range(5, C, CCHUNK):
            xc = p_ref[b, c0 : c0 + CCHUNK, r0 : r0 + rpb, :]
            e2 = jnp.exp(-xc)
            t = 1.0 + e2
            r = 1.0 / t
            sp = xc + jnp.log(t)
            facc = facc + jnp.sum(sp * r * r, axis=0)
            cm = pid_b == (ciota + c0)
            xpos = xpos + jnp.sum(jnp.where(cm, xc, 0.0), axis=0)

    e = jnp.exp(-jnp.abs(xpos))
    lg = jnp.log1p(e)
    relu = jnp.maximum(xpos, 0.0)
    p = jax.nn.sigmoid(xpos)
    ce1 = relu - xpos + lg
    om = 1.0 - p
    f1 = 0.25 * ce1 * om * om
    ce0 = relu + lg
    q = 1.0 - om
    f0 = 0.75 * ce0 * q * q
    lcls_cells = 0.75 * facc + jnp.where(keep, f1 - f0, 0.0)
    if seg["ncells"] % (rpb * 128):
        valid = cflat < seg["ncells"]
        lcls_cells = jnp.where(valid, lcls_cells, 0.0)

    xo = ps[4]
    ce = jnp.maximum(xo, 0.0) - xo * tcnt + jnp.log1p(jnp.exp(-jnp.abs(xo)))
    lcnt_cells = jnp.where(keep, ce, 0.0)

    px0 = gx - ps[0] * s
    py0 = gy - ps[1] * s
    px1 = gx + ps[2] * s
    py1 = gy + ps[3] * s
    ix0 = jnp.maximum(px0, tx0)
    iy0 = jnp.maximum(py0, ty0)
    ix1 = jnp.minimum(px1, tx1)
    iy1 = jnp.minimum(py1, ty1)
    inter = jnp.clip(ix1 - ix0, 0.0) * jnp.clip(iy1 - iy0, 0.0)
    a1 = (px1 - px0) * (py1 - py0)
    a2 = (tx1 - tx0) * (ty1 - ty0)
    union = a1 + a2 - inter + 1e-9
    iou = inter / union
    cx0 = jnp.minimum(px0, tx0)
    cy0 = jnp.minimum(py0, ty0)
    cx1 = jnp.maximum(px1, tx1)
    cy1 = jnp.maximum(py1, ty1)
    cc = (cx1 - cx0) * (cy1 - cy0) + 1e-9
    giou = iou - (cc - union) / cc
    lbox_cells = jnp.where(keep, 1.0 - giou, 0.0)

    zeros = jnp.zeros((128,), f32)
    out = jnp.stack(
        [
            jnp.sum(lbox_cells, axis=0),
            jnp.sum(lcnt_cells, axis=0),
            jnp.sum(lcls_cells, axis=0),
            jnp.sum(keep.astype(f32), axis=0),
            zeros,
            zeros,
            zeros,
            zeros,
        ],
        axis=0,
    )
    return out


_LV3 = _mk_level(0, 128, 0, 16)
_LV4 = _mk_level(1, 64, 0, 16)
_LV5 = _mk_level(2, 32, 0, 8)
_LV6 = _mk_level(3, 16, 8, 2)
_LV7 = _mk_level(4, 8, 10, 1)


def _fused_kernel(t_ref, p3_ref, pr_ref, o_ref, fidx_ref, fcnt_ref):
    i = pl.program_id(0)

    @pl.when(i < 8)
    def _():
        o_ref[...] = _level_body(t_ref, p3_ref, fidx_ref, fcnt_ref, i, _LV3)[
            None
        ]

    @pl.when((i >= 8) & (i < 10))
    def _():
        o_ref[...] = _level_body(
            t_ref, pr_ref, fidx_ref, fcnt_ref, i - 8, _LV4
        )[None]

    @pl.when(i == 10)
    def _():
        z = jnp.int32(0)
        out = (
            _level_body(t_ref, pr_ref, fidx_ref, fcnt_ref, z, _LV5)
            + _level_body(t_ref, pr_ref, fidx_ref, fcnt_ref, z, _LV6)
            + _level_body(t_ref, pr_ref, fidx_ref, fcnt_ref, z, _LV7)
        )
        o_ref[...] = out[None]


def _fcos_loss_pallas(p3, p4, p5, p6, p7, targets, interpret=False):
    tg = jnp.asarray(targets, jnp.float32)
    p7h = p7.reshape(B, C, 1, 64)
    prest = jnp.concatenate(
        [
            p4.reshape(B, C, 32, 128),
            p5.reshape(B, C, 8, 128),
            p6.reshape(B, C, 2, 128),
            jnp.concatenate([p7h, jnp.zeros_like(p7h)], axis=-1),
            jnp.zeros((B, C, 5, 128), jnp.float32),
        ],
        axis=2,
    )
    out = pl.pallas_call(
        _fused_kernel,
        grid=(_NSTEPS,),
        in_specs=[
            pl.BlockSpec(memory_space=pltpu.SMEM),
            pl.BlockSpec((B, C, 16, 128), lambda i: (0, 0, jnp.clip(i, 0, 7), 0)),
            pl.BlockSpec(
                (B, C, 16, 128), lambda i: (0, 0, jnp.clip(i - 8, 0, 2), 0)
            ),
        ],
        out_specs=pl.BlockSpec((1, 8, 128), lambda i: (i, 0, 0)),
        out_shape=jax.ShapeDtypeStruct((_NSTEPS, 8, 128), jnp.float32),
        scratch_shapes=[
            pltpu.SMEM((5, NT), jnp.int32),
            pltpu.SMEM((5,), jnp.int32),
        ],
        interpret=interpret,
    )(tg, p3, prest)
    acc = jnp.sum(out[:, :4, :], axis=(0, 2))
    n = acc[3]
    lbox = acc[0] / n
    lcnt = acc[1] / n
    lcls = acc[2] / n
    loss = lbox + lcnt + lcls
    return (loss, lbox, lcnt, lcls)


def kernel(p3, p4, p5, p6, p7, targets, image_size):
    return _fcos_loss_pallas(p3, p4, p5, p6, p7, targets)


# speedup vs baseline: 1.5298x; 1.5298x over previous
import jax
import jax.numpy as jnp
from jax.experimental import pallas as pl
from jax.experimental.pallas import tpu as pltpu

B = 8
C = 85
NT = 64
SIZES = (8.0, 16.0, 32.0, 64.0, 128.0)
GRIDS = (128, 64, 32, 16, 8)
IMG = 1024.0
CCHUNK = 8

_SEGS = []
_off = 0
for _li, _g in enumerate(GRIDS):
    _rows = max(_g * _g // 128, 1)
    _rpb = min(_rows, 16)
    _ns = _rows // _rpb
    _s = SIZES[_li]
    _SEGS.append(
        dict(
            off=_off,
            nsteps=_ns,
            g=_g,
            rows=_rows,
            rpb=_rpb,
            s=_s,
            stride=IMG / _g,
            lo=0.0 if _li == 0 else _s * 4.0,
            hi=float("inf") if _li == 4 else _s * 8.0,
            ncells=_g * _g,
        )
    )
    _off += _ns
_NSTEPS = _off


def _level_body(t_ref, p_ref, o_ref, fidx_ref, fcnt_ref, li, lstep, seg):
    f32 = jnp.float32
    g = seg["g"]
    rpb = seg["rpb"]
    s = seg["s"]
    stride = seg["stride"]
    lo = seg["lo"]
    hi = seg["hi"]
    log2g = g.bit_length() - 1

    @pl.when(lstep == 0)
    def _():
        cnt = jnp.int32(0)
        for t in range(NT):
            w = t_ref[0, t, 4] - t_ref[0, t, 2]
            h = t_ref[0, t, 5] - t_ref[0, t, 3]
            mwh = jnp.maximum(w, h)
            feas = None
            if lo > 0.0:
                feas = mwh > (lo - 1.0)
            if hi != float("inf"):
                c2 = mwh * 0.5 < (hi + 1.0)
                feas = c2 if feas is None else (feas & c2)
            fidx_ref[li, cnt] = jnp.int32(t)
            cnt = cnt + (jnp.int32(1) if feas is None else feas.astype(jnp.int32))
        fcnt_ref[li] = cnt

    rowi = jax.lax.broadcasted_iota(jnp.int32, (rpb, 128), 0)
    lane = jax.lax.broadcasted_iota(jnp.int32, (rpb, 128), 1)
    cflat = (lstep * rpb + rowi) * 128 + lane
    ix = (cflat & (g - 1)).astype(f32)
    iy = jax.lax.shift_right_logical(cflat, log2g).astype(f32)
    x = (ix + 0.5) * stride
    y = (iy + 0.5) * stride

    radius = s * 2.0
    shape = (rpb, 128)

    enc0 = t_ref[0, 0, 0] * 256.0 + t_ref[0, 0, 1]
    init = (
        jnp.zeros(shape, f32),
        jnp.full(shape, enc0, f32),
        jnp.full(shape, t_ref[0, 0, 2], f32),
        jnp.full(shape, t_ref[0, 0, 3], f32),
        jnp.full(shape, t_ref[0, 0, 4], f32),
        jnp.full(shape, t_ref[0, 0, 5], f32),
    )

    def tbody(k, carry):
        best, enc, ax0, ay0, ax1, ay1 = carry
        t = fidx_ref[li, k]
        nb = t_ref[0, t, 0]
        cl = t_ref[0, t, 1]
        x0 = t_ref[0, t, 2]
        y0 = t_ref[0, t, 3]
        x1 = t_ref[0, t, 4]
        y1 = t_ref[0, t, 5]
        l = x - x0
        tt = y - y0
        r = x1 - x
        bb = y1 - y
        omin = jnp.minimum(jnp.minimum(l, tt), jnp.minimum(r, bb))
        omax = jnp.maximum(jnp.maximum(l, tt), jnp.maximum(r, bb))
        cxb = (x0 + x1) / 2.0
        cyb = (y0 + y1) / 2.0
        cmax = jnp.maximum(jnp.abs(x - cxb), jnp.abs(y - cyb))
        jc = (omin > 0.0) & (cmax < radius)
        if lo > 0.0:
            jc = jc & (omax > lo)
        if hi != float("inf"):
            jc = jc & (omax < hi)
        area = (l + r) * (tt + bb)
        score = jnp.where(jc, 1e8 - area, 0.0)
        upd = score > best
        best = jnp.where(upd, score, best)
        enc = jnp.where(upd, nb * 256.0 + cl, enc)
        ax0 = jnp.where(upd, x0, ax0)
        ay0 = jnp.where(upd, y0, ay0)
        ax1 = jnp.where(upd, x1, ax1)
        ay1 = jnp.where(upd, y1, ay1)
        return (best, enc, ax0, ay0, ax1, ay1)

    best, enc, sx0, sy0, sx1, sy1 = jax.lax.fori_loop(
        0, fcnt_ref[li], tbody, init
    )
    keep = best > 0.0
    benc = enc.astype(jnp.int32)
    bsel = jax.lax.shift_right_logical(benc, 8)
    csel = benc & 255
    lsel = x - sx0
    tsel = y - sy0
    rsel = sx1 - x
    bsel_f = sy1 - y
    gx = lsel + sx0
    gy = tsel + sy0
    lr_min = jnp.minimum(lsel, rsel)
    lr_max = jnp.maximum(lsel, rsel)
    tb_min = jnp.minimum(tsel, bsel_f)
    tb_max = jnp.maximum(tsel, bsel_f)
    tcnt = jnp.sqrt(lr_min * tb_min / (lr_max * tb_max + 1e-10))
    tx0 = gx - lsel
    ty0 = gy - tsel
    tx1 = gx + rsel
    ty1 = gy + bsel_f

    pid = jnp.where(keep, bsel * C + (csel + 5), -1)

    facc = jnp.zeros((rpb, 128), f32)
    xpos = jnp.zeros((rpb, 128), f32)
    ps = [jnp.zeros((rpb, 128), f32) for _ in range(5)]
    ciota = jax.lax.broadcasted_iota(jnp.int32, (CCHUNK, 1, 1), 0)
    for b in range(B):
        bm = bsel == b
        for c in range(5):
            ps[c] = ps[c] + jnp.where(bm, p_ref[b, c], 0.0)
        pid_b = (pid - b * C)[None, :, :]
        for c0 in range(5, C, CCHUNK):
            xc = p_ref[b, c0 : c0 + CCHUNK]
            e2 = jnp.exp(-xc)
            t = 1.0 + e2
            r = 1.0 / t
            sp = xc + jnp.log(t)
            facc = facc + jnp.sum(sp * r * r, axis=0)
            cm = pid_b == (ciota + c0)
            xpos = xpos + jnp.sum(jnp.where(cm, xc, 0.0), axis=0)

    e = jnp.exp(-jnp.abs(xpos))
    lg = jnp.log1p(e)
    relu = jnp.maximum(xpos, 0.0)
    p = jax.nn.sigmoid(xpos)
    ce1 = relu - xpos + lg
    om = 1.0 - p
    f1 = 0.25 * ce1 * om * om
    ce0 = relu + lg
    q = 1.0 - om
    f0 = 0.75 * ce0 * q * q
    lcls_cells = 0.75 * facc + jnp.where(keep, f1 - f0, 0.0)
    if seg["ncells"] < seg["rows"] * 128:
        valid = cflat < seg["ncells"]
        lcls_cells = jnp.where(valid, lcls_cells, 0.0)

    xo = ps[4]
    ce = jnp.maximum(xo, 0.0) - xo * tcnt + jnp.log1p(jnp.exp(-jnp.abs(xo)))
    lcnt_cells = jnp.where(keep, ce, 0.0)

    px0 = gx - ps[0] * s
    py0 = gy - ps[1] * s
    px1 = gx + ps[2] * s
    py1 = gy + ps[3] * s
    ix0 = jnp.maximum(px0, tx0)
    iy0 = jnp.maximum(py0, ty0)
    ix1 = jnp.minimum(px1, tx1)
    iy1 = jnp.minimum(py1, ty1)
    inter = jnp.clip(ix1 - ix0, 0.0) * jnp.clip(iy1 - iy0, 0.0)
    a1 = (px1 - px0) * (py1 - py0)
    a2 = (tx1 - tx0) * (ty1 - ty0)
    union = a1 + a2 - inter + 1e-9
    iou = inter / union
    cx0 = jnp.minimum(px0, tx0)
    cy0 = jnp.minimum(py0, ty0)
    cx1 = jnp.maximum(px1, tx1)
    cy1 = jnp.maximum(py1, ty1)
    cc = (cx1 - cx0) * (cy1 - cy0) + 1e-9
    giou = iou - (cc - union) / cc
    lbox_cells = jnp.where(keep, 1.0 - giou, 0.0)

    zeros = jnp.zeros((128,), f32)
    out = jnp.stack(
        [
            jnp.sum(lbox_cells, axis=0),
            jnp.sum(lcnt_cells, axis=0),
            jnp.sum(lcls_cells, axis=0),
            jnp.sum(keep.astype(f32), axis=0),
            zeros,
            zeros,
            zeros,
            zeros,
        ],
        axis=0,
    )
    o_ref[...] = out[None]


def _fused_kernel(
    t_ref, p3_ref, p4_ref, p5_ref, p6_ref, p7_ref, o_ref, fidx_ref, fcnt_ref
):
    i = pl.program_id(0)
    prefs = (p3_ref, p4_ref, p5_ref, p6_ref, p7_ref)
    for li, seg in enumerate(_SEGS):
        cond = (i >= seg["off"]) & (i < seg["off"] + seg["nsteps"])

        @pl.when(cond)
        def _(li=li, seg=seg):
            _level_body(
                t_ref, prefs[li], o_ref, fidx_ref, fcnt_ref, li,
                i - seg["off"], seg,
            )


def _p_spec(seg):
    off = seg["off"]
    ns = seg["nsteps"]
    return pl.BlockSpec(
        (B, C, seg["rpb"], 128),
        lambda i, off=off, ns=ns: (0, 0, jnp.clip(i - off, 0, ns - 1), 0),
    )


def _fcos_loss_pallas(p3, p4, p5, p6, p7, targets, interpret=False):
    tg = jnp.asarray(targets, jnp.float32)
    p4r = p4.reshape(B, C, 32, 128)
    p5r = p5.reshape(B, C, 8, 128)
    p6r = p6.reshape(B, C, 2, 128)
    p7h = p7.reshape(B, C, 1, 64)
    p7r = jnp.concatenate([p7h, jnp.zeros_like(p7h)], axis=-1)
    out = pl.pallas_call(
        _fused_kernel,
        grid=(_NSTEPS,),
        in_specs=[pl.BlockSpec(memory_space=pltpu.SMEM)]
        + [_p_spec(seg) for seg in _SEGS],
        out_specs=pl.BlockSpec((1, 8, 128), lambda i: (i, 0, 0)),
        out_shape=jax.ShapeDtypeStruct((_NSTEPS, 8, 128), jnp.float32),
        scratch_shapes=[
            pltpu.SMEM((5, NT), jnp.int32),
            pltpu.SMEM((5,), jnp.int32),
        ],
        interpret=interpret,
    )(tg, p3, p4r, p5r, p6r, p7r)
    acc = jnp.sum(out[:, :4, :], axis=(0, 2))
    n = acc[3]
    lbox = acc[0] / n
    lcnt = acc[1] / n
    lcls = acc[2] / n
    loss = lbox + lcnt + lcls
    return (loss, lbox, lcnt, lcls)


def kernel(p3, p4, p5, p6, p7, targets, image_size):
    return _fcos_loss_pallas(p3, p4, p5, p6, p7, targets)


# speedup vs baseline: 1.5320x; 1.0015x over previous
import jax
import jax.numpy as jnp
from jax.experimental import pallas as pl
from jax.experimental.pallas import tpu as pltpu

B = 8
C = 85
NT = 64
SIZES = (8.0, 16.0, 32.0, 64.0, 128.0)
GRIDS = (128, 64, 32, 16, 8)
IMG = 1024.0
CCHUNK = 8

_SEGS = []
_off = 0
for _li, _g in enumerate(GRIDS):
    _rows = max(_g * _g // 128, 1)
    _rpb = min(_rows, 16)
    _ns = _rows // _rpb
    _s = SIZES[_li]
    _SEGS.append(
        dict(
            off=_off,
            nsteps=_ns,
            g=_g,
            rows=_rows,
            rpb=_rpb,
            s=_s,
            stride=IMG / _g,
            lo=0.0 if _li == 0 else _s * 4.0,
            hi=float("inf") if _li == 4 else _s * 8.0,
            ncells=_g * _g,
        )
    )
    _off += _ns
_NSTEPS = _off


def _level_body(t_ref, p_ref, o_ref, fidx_ref, fcnt_ref, li, lstep, seg):
    f32 = jnp.float32
    g = seg["g"]
    rpb = seg["rpb"]
    s = seg["s"]
    stride = seg["stride"]
    lo = seg["lo"]
    hi = seg["hi"]
    log2g = g.bit_length() - 1

    @pl.when(lstep == 0)
    def _():
        cnt = jnp.int32(0)
        for t in range(NT):
            w = t_ref[0, t, 4] - t_ref[0, t, 2]
            h = t_ref[0, t, 5] - t_ref[0, t, 3]
            mwh = jnp.maximum(w, h)
            feas = None
            if lo > 0.0:
                feas = mwh > (lo - 1.0)
            if hi != float("inf"):
                c2 = mwh * 0.5 < (hi + 1.0)
                feas = c2 if feas is None else (feas & c2)
            fidx_ref[li, cnt] = jnp.int32(t)
            cnt = cnt + (jnp.int32(1) if feas is None else feas.astype(jnp.int32))
        fcnt_ref[li] = cnt

    rowi = jax.lax.broadcasted_iota(jnp.int32, (rpb, 128), 0)
    lane = jax.lax.broadcasted_iota(jnp.int32, (rpb, 128), 1)
    cflat = (lstep * rpb + rowi) * 128 + lane
    ix = (cflat & (g - 1)).astype(f32)
    iy = jax.lax.shift_right_logical(cflat, log2g).astype(f32)
    x = (ix + 0.5) * stride
    y = (iy + 0.5) * stride

    radius = s * 2.0
    shape = (rpb, 128)

    enc0 = t_ref[0, 0, 0] * 256.0 + t_ref[0, 0, 1]
    init = (
        jnp.zeros(shape, f32),
        jnp.full(shape, enc0, f32),
        jnp.full(shape, t_ref[0, 0, 2], f32),
        jnp.full(shape, t_ref[0, 0, 3], f32),
        jnp.full(shape, t_ref[0, 0, 4], f32),
        jnp.full(shape, t_ref[0, 0, 5], f32),
    )

    def tbody(k, carry):
        best, enc, ax0, ay0, ax1, ay1 = carry
        t = fidx_ref[li, k]
        nb = t_ref[0, t, 0]
        cl = t_ref[0, t, 1]
        x0 = t_ref[0, t, 2]
        y0 = t_ref[0, t, 3]
        x1 = t_ref[0, t, 4]
        y1 = t_ref[0, t, 5]
        l = x - x0
        tt = y - y0
        r = x1 - x
        bb = y1 - y
        omin = jnp.minimum(jnp.minimum(l, tt), jnp.minimum(r, bb))
        omax = jnp.maximum(jnp.maximum(l, tt), jnp.maximum(r, bb))
        cxb = (x0 + x1) / 2.0
        cyb = (y0 + y1) / 2.0
        cmax = jnp.maximum(jnp.abs(x - cxb), jnp.abs(y - cyb))
        jc = (omin > 0.0) & (cmax < radius)
        if lo > 0.0:
            jc = jc & (omax > lo)
        if hi != float("inf"):
            jc = jc & (omax < hi)
        area = (l + r) * (tt + bb)
        score = jnp.where(jc, 1e8 - area, 0.0)
        upd = score > best
        best = jnp.where(upd, score, best)
        enc = jnp.where(upd, nb * 256.0 + cl, enc)
        ax0 = jnp.where(upd, x0, ax0)
        ay0 = jnp.where(upd, y0, ay0)
        ax1 = jnp.where(upd, x1, ax1)
        ay1 = jnp.where(upd, y1, ay1)
        return (best, enc, ax0, ay0, ax1, ay1)

    best, enc, sx0, sy0, sx1, sy1 = jax.lax.fori_loop(
        0, fcnt_ref[li], tbody, init
    )
    keep = best > 0.0
    benc = enc.astype(jnp.int32)
    bsel = jax.lax.shift_right_logical(benc, 8)
    csel = benc & 255
    lsel = x - sx0
    tsel = y - sy0
    rsel = sx1 - x
    bsel_f = sy1 - y
    gx = lsel + sx0
    gy = tsel + sy0
    lr_min = jnp.minimum(lsel, rsel)
    lr_max = jnp.maximum(lsel, rsel)
    tb_min = jnp.minimum(tsel, bsel_f)
    tb_max = jnp.maximum(tsel, bsel_f)
    tcnt = jnp.sqrt(lr_min * tb_min / (lr_max * tb_max + 1e-10))
    tx0 = gx - lsel
    ty0 = gy - tsel
    tx1 = gx + rsel
    ty1 = gy + bsel_f

    pid = jnp.where(keep, bsel * C + (csel + 5), -1)

    facc = jnp.zeros((rpb, 128), f32)
    xpos = jnp.zeros((rpb, 128), f32)
    ps = [jnp.zeros((rpb, 128), f32) for _ in range(5)]
    ciota = jax.lax.broadcasted_iota(jnp.int32, (CCHUNK, 1, 1), 0)
    for b in range(B):
        bm = bsel == b
        for c in range(5):
            ps[c] = ps[c] + jnp.where(bm, p_ref[b, c], 0.0)
        pid_b = (pid - b * C)[None, :, :]
        for c0 in range(5, C, CCHUNK):
            xc = p_ref[b, c0 : c0 + CCHUNK]
            e2 = jnp.exp(-xc)
            t = 1.0 + e2
            r = pl.reciprocal(t, approx=True)
            sp = xc + jnp.log(t)
            facc = facc + jnp.sum(sp * r * r, axis=0)
            cm = pid_b == (ciota + c0)
            xpos = xpos + jnp.sum(jnp.where(cm, xc, 0.0), axis=0)

    e = jnp.exp(-jnp.abs(xpos))
    lg = jnp.log1p(e)
    relu = jnp.maximum(xpos, 0.0)
    p = jax.nn.sigmoid(xpos)
    ce1 = relu - xpos + lg
    om = 1.0 - p
    f1 = 0.25 * ce1 * om * om
    ce0 = relu + lg
    q = 1.0 - om
    f0 = 0.75 * ce0 * q * q
    lcls_cells = 0.75 * facc + jnp.where(keep, f1 - f0, 0.0)
    if seg["ncells"] < seg["rows"] * 128:
        valid = cflat < seg["ncells"]
        lcls_cells = jnp.where(valid, lcls_cells, 0.0)

    xo = ps[4]
    ce = jnp.maximum(xo, 0.0) - xo * tcnt + jnp.log1p(jnp.exp(-jnp.abs(xo)))
    lcnt_cells = jnp.where(keep, ce, 0.0)

    px0 = gx - ps[0] * s
    py0 = gy - ps[1] * s
    px1 = gx + ps[2] * s
    py1 = gy + ps[3] * s
    ix0 = jnp.maximum(px0, tx0)
    iy0 = jnp.maximum(py0, ty0)
    ix1 = jnp.minimum(px1, tx1)
    iy1 = jnp.minimum(py1, ty1)
    inter = jnp.clip(ix1 - ix0, 0.0) * jnp.clip(iy1 - iy0, 0.0)
    a1 = (px1 - px0) * (py1 - py0)
    a2 = (tx1 - tx0) * (ty1 - ty0)
    union = a1 + a2 - inter + 1e-9
    iou = inter / union
    cx0 = jnp.minimum(px0, tx0)
    cy0 = jnp.minimum(py0, ty0)
    cx1 = jnp.maximum(px1, tx1)
    cy1 = jnp.maximum(py1, ty1)
    cc = (cx1 - cx0) * (cy1 - cy0) + 1e-9
    giou = iou - (cc - union) / cc
    lbox_cells = jnp.where(keep, 1.0 - giou, 0.0)

    zeros = jnp.zeros((128,), f32)
    out = jnp.stack(
        [
            jnp.sum(lbox_cells, axis=0),
            jnp.sum(lcnt_cells, axis=0),
            jnp.sum(lcls_cells, axis=0),
            jnp.sum(keep.astype(f32), axis=0),
            zeros,
            zeros,
            zeros,
            zeros,
        ],
        axis=0,
    )
    o_ref[...] = out[None]


def _fused_kernel(
    t_ref, p3_ref, p4_ref, p5_ref, p6_ref, p7_ref, o_ref, fidx_ref, fcnt_ref
):
    i = pl.program_id(0)
    prefs = (p3_ref, p4_ref, p5_ref, p6_ref, p7_ref)
    for li, seg in enumerate(_SEGS):
        cond = (i >= seg["off"]) & (i < seg["off"] + seg["nsteps"])

        @pl.when(cond)
        def _(li=li, seg=seg):
            _level_body(
                t_ref, prefs[li], o_ref, fidx_ref, fcnt_ref, li,
                i - seg["off"], seg,
            )


def _p_spec(seg):
    off = seg["off"]
    ns = seg["nsteps"]
    return pl.BlockSpec(
        (B, C, seg["rpb"], 128),
        lambda i, off=off, ns=ns: (0, 0, jnp.clip(i - off, 0, ns - 1), 0),
    )


def _fcos_loss_pallas(p3, p4, p5, p6, p7, targets, interpret=False):
    tg = jnp.asarray(targets, jnp.float32)
    p4r = p4.reshape(B, C, 32, 128)
    p5r = p5.reshape(B, C, 8, 128)
    p6r = p6.reshape(B, C, 2, 128)
    p7h = p7.reshape(B, C, 1, 64)
    p7r = jnp.concatenate([p7h, jnp.zeros_like(p7h)], axis=-1)
    out = pl.pallas_call(
        _fused_kernel,
        grid=(_NSTEPS,),
        in_specs=[pl.BlockSpec(memory_space=pltpu.SMEM)]
        + [_p_spec(seg) for seg in _SEGS],
        out_specs=pl.BlockSpec((1, 8, 128), lambda i: (i, 0, 0)),
        out_shape=jax.ShapeDtypeStruct((_NSTEPS, 8, 128), jnp.float32),
        scratch_shapes=[
            pltpu.SMEM((5, NT), jnp.int32),
            pltpu.SMEM((5,), jnp.int32),
        ],
        interpret=interpret,
    )(tg, p3, p4r, p5r, p6r, p7r)
    acc = jnp.sum(out[:, :4, :], axis=(0, 2))
    n = acc[3]
    lbox = acc[0] / n
    lcnt = acc[1] / n
    lcls = acc[2] / n
    loss = lbox + lcnt + lcls
    return (loss, lbox, lcnt, lcls)


def kernel(p3, p4, p5, p6, p7, targets, image_size):
    return _fcos_loss_pallas(p3, p4, p5, p6, p7, targets)
